# baseline (speedup 1.0000x reference)
import jax
import jax.numpy as jnp
from jax.experimental import pallas as pl
from jax.experimental.pallas import tpu as pltpu

TAU = 0.07
EPS = 1e-12

B = 4
C = 512
K = 21
HW = 64 * 64
P = B * HW
NI = 8
T_P = P // NI


def _body(x_ref, gt_ref, out_ref, xb_scr, rn_scr, m_scr, k0_scr, k0n_scr,
          acc_ref):
    i = pl.program_id(1)

    @pl.when((pl.program_id(0) == 0) & (i == 0))
    def _init():
        k0_scr[...] = jnp.zeros_like(k0_scr)
        acc_ref[0] = 0.0
        acc_ref[1] = 0.0
        acc_ref[2] = 0.0

    @pl.when(pl.program_id(0) == 0)
    def _phase0():
        x = x_ref[...]
        xb = x.astype(jnp.bfloat16)
        xb_scr[pl.ds(i, 1)] = xb[None]
        ones = jnp.ones((8, C), jnp.bfloat16)
        s2row = jax.lax.dot_general(
            ones, xb * xb,
            dimension_numbers=(((1,), (1,)), ((), ())),
            preferred_element_type=jnp.float32)[0:1]
        rnorm = 1.0 / jnp.maximum(jnp.sqrt(s2row), EPS)
        rn_scr[pl.ds(i, 1)] = rnorm[None]
        maskf = (gt_ref[0] == jnp.uint8(1)).astype(jnp.float32)
        m_scr[pl.ds(i, 1)] = jnp.sum(maskf, axis=0, keepdims=True)[None]
        wmask = (maskf * rnorm).astype(jnp.bfloat16)
        k0_scr[...] += jax.lax.dot_general(
            wmask, xb,
            dimension_numbers=(((1,), (0,)), ((), ())),
            preferred_element_type=jnp.float32)
        acc_ref[0] += jnp.sum(maskf)

        @pl.when(i == NI - 1)
        def _finalize_k0():
            k0 = k0_scr[...]
            s = jnp.sum(k0 * k0, axis=1, keepdims=True)
            nrm = jnp.sqrt(s)
            k0n_scr[...] = (k0 / jnp.maximum(nrm, EPS)).astype(jnp.bfloat16)
            acc_ref[1] = jnp.sum(s / jnp.maximum(nrm, EPS)) * (1.0 / TAU)

    @pl.when(pl.program_id(0) == 1)
    def _phase1():
        xb = xb_scr[i]
        simraw = jax.lax.dot_general(
            k0n_scr[...], xb,
            dimension_numbers=(((1,), (1,)), ((), ())),
            preferred_element_type=jnp.float32)
        sim = simraw * (rn_scr[i] * (1.0 / TAU))
        mx = jnp.max(sim, axis=0, keepdims=True)
        lse = mx + jnp.log(jnp.sum(jnp.exp(sim - mx), axis=0, keepdims=True))
        acc_ref[2] += jnp.sum(m_scr[i] * lse)

        @pl.when(i == NI - 1)
        def _final():
            out_ref[...] = jnp.broadcast_to(
                (acc_ref[2] - acc_ref[1]) / acc_ref[0], (1, 1))


def kernel(feat, gt):
    x = feat.transpose(0, 2, 3, 1).reshape(P, C)
    gt2 = gt.astype(jnp.uint8).reshape(B, K, HW)
    out = pl.pallas_call(
        _body,
        grid=(2, NI),
        in_specs=[
            pl.BlockSpec((T_P, C),
                         lambda p, i: (jnp.where(p == 0, i, NI - 1), 0)),
            pl.BlockSpec((1, K, T_P), lambda p, i: (i // 2, 0, i % 2)),
        ],
        out_specs=pl.BlockSpec((1, 1), lambda p, i: (0, 0)),
        out_shape=jax.ShapeDtypeStruct((1, 1), jnp.float32),
        scratch_shapes=[
            pltpu.VMEM((NI, T_P, C), jnp.bfloat16),
            pltpu.VMEM((NI, 1, T_P), jnp.float32),
            pltpu.VMEM((NI, 1, T_P), jnp.float32),
            pltpu.VMEM((K, C), jnp.float32),
            pltpu.VMEM((K, C), jnp.bfloat16),
            pltpu.SMEM((3,), jnp.float32),
        ],
    )(x, gt2)
    return out.reshape(1)


# speedup vs baseline: 1.1426x; 1.1426x over previous
import jax
import jax.numpy as jnp
from jax.experimental import pallas as pl
from jax.experimental.pallas import tpu as pltpu

TAU = 0.07
EPS = 1e-12

B = 4
C = 512
K = 21
HW = 64 * 64
P = B * HW
NI = 4
T_P = P // NI


def _body(x_ref, gt_ref, out_ref, xb_scr, rn_scr, m_scr, k0_scr, k0n_scr,
          acc_ref):
    i = pl.program_id(1)

    @pl.when((pl.program_id(0) == 0) & (i == 0))
    def _init():
        k0_scr[...] = jnp.zeros_like(k0_scr)
        acc_ref[0] = 0.0
        acc_ref[1] = 0.0
        acc_ref[2] = 0.0

    @pl.when(pl.program_id(0) == 0)
    def _phase0():
        x = x_ref[...]
        xb = x.astype(jnp.bfloat16)
        xb_scr[pl.ds(i, 1)] = xb[None]
        ones = jnp.ones((8, C), jnp.bfloat16)
        s2row = jax.lax.dot_general(
            ones, xb * xb,
            dimension_numbers=(((1,), (1,)), ((), ())),
            preferred_element_type=jnp.float32)[0:1]
        rnorm = 1.0 / jnp.maximum(jnp.sqrt(s2row), EPS)
        rn_scr[pl.ds(i, 1)] = rnorm[None]
        maskf = (gt_ref[0] == 1).astype(jnp.float32)
        m_scr[pl.ds(i, 1)] = jnp.sum(maskf, axis=0, keepdims=True)[None]
        wmask = (maskf * rnorm).astype(jnp.bfloat16)
        k0_scr[...] += jax.lax.dot_general(
            wmask, xb,
            dimension_numbers=(((1,), (0,)), ((), ())),
            preferred_element_type=jnp.float32)
        acc_ref[0] += jnp.sum(maskf)

        @pl.when(i == NI - 1)
        def _finalize_k0():
            k0 = k0_scr[...]
            s = jnp.sum(k0 * k0, axis=1, keepdims=True)
            nrm = jnp.sqrt(s)
            k0n_scr[...] = (k0 / jnp.maximum(nrm, EPS)).astype(jnp.bfloat16)
            acc_ref[1] = jnp.sum(s / jnp.maximum(nrm, EPS)) * (1.0 / TAU)

    @pl.when(pl.program_id(0) == 1)
    def _phase1():
        xb = xb_scr[i]
        simraw = jax.lax.dot_general(
            k0n_scr[...], xb,
            dimension_numbers=(((1,), (1,)), ((), ())),
            preferred_element_type=jnp.float32)
        sim = simraw * (rn_scr[i] * (1.0 / TAU))
        mx = jnp.max(sim, axis=0, keepdims=True)
        lse = mx + jnp.log(jnp.sum(jnp.exp(sim - mx), axis=0, keepdims=True))
        acc_ref[2] += jnp.sum(m_scr[i] * lse)

        @pl.when(i == NI - 1)
        def _final():
            out_ref[...] = jnp.broadcast_to(
                (acc_ref[2] - acc_ref[1]) / acc_ref[0], (1, 1))


def kernel(feat, gt):
    x = feat.transpose(0, 2, 3, 1).reshape(P, C)
    gt2 = gt.reshape(B, K, HW)
    out = pl.pallas_call(
        _body,
        grid=(2, NI),
        in_specs=[
            pl.BlockSpec((T_P, C),
                         lambda p, i: (jnp.where(p == 0, i, NI - 1), 0)),
            pl.BlockSpec((1, K, T_P), lambda p, i: (jnp.where(p == 0, i, NI - 1), 0, 0)),
        ],
        out_specs=pl.BlockSpec((1, 1), lambda p, i: (0, 0)),
        out_shape=jax.ShapeDtypeStruct((1, 1), jnp.float32),
        scratch_shapes=[
            pltpu.VMEM((NI, T_P, C), jnp.bfloat16),
            pltpu.VMEM((NI, 1, T_P), jnp.float32),
            pltpu.VMEM((NI, 1, T_P), jnp.float32),
            pltpu.VMEM((K, C), jnp.float32),
            pltpu.VMEM((K, C), jnp.bfloat16),
            pltpu.SMEM((3,), jnp.float32),
        ],
    )(x, gt2)
    return out.reshape(1)


# speedup vs baseline: 1.2003x; 1.0505x over previous
import jax
import jax.numpy as jnp
from jax.experimental import pallas as pl
from jax.experimental.pallas import tpu as pltpu

TAU = 0.07
EPS = 1e-12

B = 4
C = 512
K = 21
HW = 64 * 64
P = B * HW
NI = 4
T_P = P // NI


def _body(x_ref, gt_ref, out_ref, xb_scr, rn_scr, m_scr, k0_scr, acc_ref):
    i = pl.program_id(0)

    @pl.when(i == 0)
    def _init():
        k0_scr[...] = jnp.zeros_like(k0_scr)
        acc_ref[0] = 0.0

    x = x_ref[...]
    xb = x.astype(jnp.bfloat16)
    xb_scr[pl.ds(i, 1)] = xb[None]
    ones = jnp.ones((8, C), jnp.bfloat16)
    s2row = jax.lax.dot_general(
        ones, xb * xb,
        dimension_numbers=(((1,), (1,)), ((), ())),
        preferred_element_type=jnp.float32)[0:1]
    rnorm = 1.0 / jnp.maximum(jnp.sqrt(s2row), EPS)
    rn_scr[pl.ds(i, 1)] = rnorm[None]
    maskf = (gt_ref[0] == 1).astype(jnp.float32)
    m_scr[pl.ds(i, 1)] = jnp.sum(maskf, axis=0, keepdims=True)[None]
    wmask = (maskf * rnorm).astype(jnp.bfloat16)
    k0_scr[...] += jax.lax.dot_general(
        wmask, xb,
        dimension_numbers=(((1,), (0,)), ((), ())),
        preferred_element_type=jnp.float32)
    acc_ref[0] += jnp.sum(maskf)

    @pl.when(i == NI - 1)
    def _contrast():
        k0 = k0_scr[...]
        s = jnp.sum(k0 * k0, axis=1, keepdims=True)
        nrm = jnp.maximum(jnp.sqrt(s), EPS)
        k0n = (k0 / nrm).astype(jnp.bfloat16)
        pos_term = jnp.sum(s / nrm) * (1.0 / TAU)
        mlse = 0.0
        for t in range(NI):
            simraw = jax.lax.dot_general(
                k0n, xb_scr[t],
                dimension_numbers=(((1,), (1,)), ((), ())),
                preferred_element_type=jnp.float32)
            sim = simraw * (rn_scr[t] * (1.0 / TAU))
            mx = jnp.max(sim, axis=0, keepdims=True)
            lse = mx + jnp.log(
                jnp.sum(jnp.exp(sim - mx), axis=0, keepdims=True))
            mlse += jnp.sum(m_scr[t] * lse)
        out_ref[...] = jnp.broadcast_to(
            (mlse - pos_term) / acc_ref[0], (1, 1))


def kernel(feat, gt):
    x = feat.transpose(0, 2, 3, 1).reshape(P, C)
    gt2 = gt.reshape(B, K, HW)
    out = pl.pallas_call(
        _body,
        grid=(NI,),
        in_specs=[
            pl.BlockSpec((T_P, C), lambda i: (i, 0)),
            pl.BlockSpec((1, K, T_P), lambda i: (i, 0, 0)),
        ],
        out_specs=pl.BlockSpec((1, 1), lambda i: (0, 0)),
        out_shape=jax.ShapeDtypeStruct((1, 1), jnp.float32),
        scratch_shapes=[
            pltpu.VMEM((NI, T_P, C), jnp.bfloat16),
            pltpu.VMEM((NI, 1, T_P), jnp.float32),
            pltpu.VMEM((NI, 1, T_P), jnp.float32),
            pltpu.VMEM((K, C), jnp.float32),
            pltpu.SMEM((1,), jnp.float32),
        ],
    )(x, gt2)
    return out.reshape(1)
